# baseline (speedup 1.0000x reference)
import functools
import math

import jax
import jax.numpy as jnp
from jax import lax
from jax.experimental import pallas as pl
from jax.experimental.pallas import tpu as pltpu
from jax.experimental.pallas import tpu_sc as plsc

D_MODEL = 64
SEQ = 200
CHUNK = 200
IDX_MINOR = 100
GATHERS = CHUNK // IDX_MINOR
LANES = 16
NUM_WORKERS = 32
NBUF = 4
LOOKAHEAD = 2


def _positional_encoding(seq, d_model):
    position = jnp.arange(0, seq, dtype=jnp.float32)[:, None]
    div_term = 10000.0 ** (jnp.arange(0, d_model, 2, dtype=jnp.float32) / d_model)
    args = position / div_term
    pe = jnp.zeros((seq, d_model), dtype=jnp.float32)
    pe = pe.at[:, 0::2].set(jnp.sin(args))
    pe = pe.at[:, 1::2].set(jnp.cos(args))
    return pe


@functools.cache
def _build_sc_call(rows, d_model):
    rows_per_worker = rows // NUM_WORKERS
    chunks_per_worker = rows_per_worker // CHUNK
    slab_streams = rows_per_worker // IDX_MINOR
    scale = math.sqrt(d_model)
    mesh = plsc.VectorSubcoreMesh(core_axis_name="c", subcore_axis_name="s")

    @functools.partial(
        pl.kernel,
        mesh=mesh,
        compiler_params=pltpu.CompilerParams(use_tc_tiling_on_sc=False),
        out_type=jax.ShapeDtypeStruct((rows, d_model), jnp.float32),
        scratch_types=[
            pltpu.VMEM((slab_streams, IDX_MINOR), jnp.int32),
            pltpu.VMEM((NBUF, CHUNK, d_model), jnp.float32),
            pltpu.VMEM((CHUNK, d_model), jnp.float32),
            [pltpu.SemaphoreType.DMA] * NBUF,
            [pltpu.SemaphoreType.DMA] * NBUF,
        ],
    )
    def sc_kernel(idx_hbm, table_hbm, pe_hbm, out_hbm, idx_v, dest_v, pe_v,
                  sem_g, sem_o):
        wid = lax.axis_index("s") * 2 + lax.axis_index("c")
        pltpu.sync_copy(pe_hbm, pe_v)
        pltpu.sync_copy(idx_hbm.at[wid], idx_v)

        def fire_gather(b, l):
            for j in range(GATHERS):
                pltpu.async_copy(
                    table_hbm.at[idx_v.at[l * GATHERS + j]],
                    dest_v.at[b, pl.ds(j * IDX_MINOR, IDX_MINOR)],
                    sem_g[b],
                )

        def wait_gather(b):
            for j in range(GATHERS):
                pltpu.make_async_copy(
                    table_hbm.at[idx_v.at[j]],
                    dest_v.at[b, pl.ds(j * IDX_MINOR, IDX_MINOR)],
                    sem_g[b],
                ).wait()

        def fire_out(b, l):
            base = (wid * chunks_per_worker + l) * CHUNK
            pltpu.async_copy(dest_v.at[b], out_hbm.at[pl.ds(base, CHUNK)],
                             sem_o[b])

        def wait_out(b):
            pltpu.make_async_copy(dest_v.at[b],
                                  out_hbm.at[pl.ds(0, CHUNK)], sem_o[b]).wait()

        for b in range(LOOKAHEAD):
            fire_gather(b, b)

        def body(c4, carry):
            for b in range(NBUF):
                l = c4 * NBUF + b
                wait_gather(b)

                def sweep(r, _):
                    for d in range(d_model // LANES):
                        sl = pl.ds(d * LANES, LANES)
                        dest_v[b, r, sl] = dest_v[b, r, sl] * scale + pe_v[r, sl]
                    return 0

                lax.fori_loop(0, CHUNK, sweep, 0)
                fire_out(b, l)
                bf = (b + LOOKAHEAD) % NBUF
                lf = l + LOOKAHEAD

                @pl.when(lf < chunks_per_worker)
                def _():
                    @pl.when(lf >= NBUF)
                    def _():
                        wait_out(bf)
                    fire_gather(bf, lf)
            return carry

        lax.fori_loop(0, chunks_per_worker // NBUF, body, 0)
        for b in range(NBUF):
            wait_out(b)

    return sc_kernel


def kernel(idxs, emb_table):
    batch, seq = idxs.shape
    vocab, d_model = emb_table.shape
    rows = batch * seq
    idx_resh = idxs.astype(jnp.int32).reshape(
        NUM_WORKERS, rows // (NUM_WORKERS * IDX_MINOR), IDX_MINOR)
    pe_tiled = jnp.tile(_positional_encoding(seq, d_model), (CHUNK // seq, 1))
    out_flat = _build_sc_call(rows, d_model)(idx_resh, emb_table, pe_tiled)
    return out_flat.reshape(batch, seq, d_model)
